# Initial kernel scaffold; baseline (speedup 1.0000x reference)
import jax
import jax.numpy as jnp
from jax.experimental import pallas as pl


def kernel(x, train_feats, y_train):
    raise NotImplementedError("write your pallas kernel here")


# speedup vs baseline: 22.7311x; 22.7311x over previous
import jax
import jax.numpy as jnp
from jax import lax
from jax.experimental import pallas as pl
from jax.experimental.pallas import tpu as pltpu

Q = 1024
N = 100000
D = 32
K_NN = 1000
C = 10

QB = 64
CW = 2048
NPAD = 100352
SEARCH_ITERS = 26


def _body(x_ref, tft_ref, oht_ref, out_ref, d2_ref):
    nchunk = NPAD // CW
    xm2 = x_ref[...] * -2.0

    def p1(j, carry):
        rmin, rmax = carry
        tft = tft_ref[:, pl.ds(j * CW, CW)]
        t2 = jnp.sum(tft * tft, axis=0, keepdims=True)
        d2 = jnp.dot(xm2, tft, preferred_element_type=jnp.float32) + t2
        col = lax.broadcasted_iota(jnp.int32, (1, CW), 1) + j * CW
        valid = col < N
        d2_ref[:, pl.ds(j * CW, CW)] = jnp.where(valid, d2, 1e30)
        rmin = jnp.minimum(
            rmin, jnp.min(jnp.where(valid, d2, 1e30), axis=1, keepdims=True))
        rmax = jnp.maximum(
            rmax, jnp.max(jnp.where(valid, d2, -1e30), axis=1, keepdims=True))
        return rmin, rmax

    init = (jnp.full((QB, 1), 1e30, jnp.float32),
            jnp.full((QB, 1), -1e30, jnp.float32))
    rmin, rmax = lax.fori_loop(0, nchunk, p1, init)

    kf = jnp.float32(K_NN)

    def bs(_, carry):
        lo, hi = carry
        mid = 0.5 * (lo + hi)

        def cnt_body(j, acc):
            d2 = d2_ref[:, pl.ds(j * CW, CW)]
            return acc + jnp.sum(jnp.where(d2 <= mid, 1.0, 0.0),
                                 axis=1, keepdims=True)

        cnt = lax.fori_loop(0, nchunk, cnt_body,
                            jnp.zeros((QB, 1), jnp.float32))
        ge = cnt >= kf
        return jnp.where(ge, lo, mid), jnp.where(ge, mid, hi)

    lo, hi = lax.fori_loop(0, SEARCH_ITERS, bs, (rmin, rmax))

    def cc(j, acc):
        d2 = d2_ref[:, pl.ds(j * CW, CW)]
        m = jnp.where(d2 <= hi, 1.0, 0.0)
        oht = oht_ref[:, pl.ds(j * CW, CW)]
        return acc + lax.dot_general(m, oht, (((1,), (1,)), ((), ())),
                                     preferred_element_type=jnp.float32)

    counts = lax.fori_loop(0, nchunk, cc, jnp.zeros((QB, 16), jnp.float32))
    total = jnp.sum(counts, axis=1, keepdims=True)
    out_ref[...] = counts / total


def kernel(x, train_feats, y_train):
    tft = jnp.zeros((D, NPAD), jnp.float32).at[:, :N].set(train_feats.T)
    y = y_train.astype(jnp.int32)
    oht = (y[None, :] == jnp.arange(16, dtype=jnp.int32)[:, None])
    oht = jnp.zeros((16, NPAD), jnp.float32).at[:, :N].set(
        oht.astype(jnp.float32))
    out = pl.pallas_call(
        _body,
        grid=(Q // QB,),
        in_specs=[
            pl.BlockSpec((QB, D), lambda q: (q, 0)),
            pl.BlockSpec((D, NPAD), lambda q: (0, 0)),
            pl.BlockSpec((16, NPAD), lambda q: (0, 0)),
        ],
        out_specs=pl.BlockSpec((QB, 16), lambda q: (q, 0)),
        out_shape=jax.ShapeDtypeStruct((Q, 16), jnp.float32),
        scratch_shapes=[pltpu.VMEM((QB, NPAD), jnp.float32)],
    )(x, tft, oht)
    return out[:, :C]
